# baseline (speedup 1.0000x reference)
import functools

import jax
import jax.numpy as jnp
from jax import lax
from jax.experimental import pallas as pl
from jax.experimental.pallas import tpu as pltpu
from jax.experimental.pallas import tpu_sc as plsc

_NUM_CORES = 2
_NUM_SUBCORES = 16
_LANES = 16
_NW = _NUM_CORES * _NUM_SUBCORES
_TILE_LANES = 128
_CHUNK = 1
_SLOTS = 8


@functools.lru_cache(maxsize=None)
def _build_gather_add(n_tokens: int, batch: int, seq_len_s: int, d: int):
    b_per_w = n_tokens // _NW
    chunks_per_row = seq_len_s // b_per_w
    n_chunks = b_per_w // _CHUNK
    group_chunks = _LANES // _CHUNK

    mesh = plsc.VectorSubcoreMesh(
        core_axis_name="c", subcore_axis_name="s",
        num_cores=_NUM_CORES, num_subcores=_NUM_SUBCORES)

    @functools.partial(
        pl.kernel,
        mesh=mesh,
        compiler_params=pltpu.CompilerParams(needs_layout_passes=False),
        out_type=jax.ShapeDtypeStruct((batch, d, seq_len_s), jnp.float32),
        scratch_types=[
            pltpu.VMEM((b_per_w,), jnp.int32),
            pltpu.VMEM((d, b_per_w), jnp.float32),
            pltpu.VMEM((d, b_per_w), jnp.float32),
            [pltpu.VMEM((_CHUNK, d, _TILE_LANES), jnp.float32)] * _SLOTS,
            [pltpu.SemaphoreType.DMA] * _SLOTS,
        ],
    )
    def gather_add(ids_hbm, table_hbm, pos_hbm, out_hbm, idx_v, pos_v,
                   outc_v, bufs, sems):
        wid = lax.axis_index("s") * _NUM_CORES + lax.axis_index("c")
        base = wid * b_per_w
        b_i = wid // chunks_per_row
        s_off = pl.multiple_of(
            lax.rem(wid, chunks_per_row) * b_per_w, _TILE_LANES)
        pltpu.sync_copy(ids_hbm.at[pl.ds(base, b_per_w)], idx_v)
        pltpu.sync_copy(pos_hbm.at[:, pl.ds(s_off, b_per_w)], pos_v)

        tok_iota = lax.iota(jnp.int32, _LANES)

        def fire_chunk(slot, vec, lane0):
            for j in range(_CHUNK):
                col0 = pl.multiple_of(
                    lax.bitwise_and(vec[lane0 + j], -_TILE_LANES),
                    _TILE_LANES)
                for fb in range(8):
                    pltpu.async_copy(
                        table_hbm.at[pl.ds(fb * 8, 8), pl.ds(col0, _TILE_LANES)],
                        bufs[slot].at[j, pl.ds(fb * 8, 8)], sems[slot])

        def wait_chunk(slot):
            for j in range(_CHUNK):
                pltpu.make_async_copy(
                    table_hbm.at[:, pl.ds(0, _TILE_LANES)],
                    bufs[slot].at[j], sems[slot]).wait()

        def process_chunk(slot, vec, lane0, t0):
            for j in range(_CHUNK):
                lane_v = jnp.full(
                    (_LANES,), lax.bitwise_and(vec[lane0 + j], _TILE_LANES - 1),
                    jnp.int32)
                j_v = jnp.full((_LANES,), j, jnp.int32)
                t_v = jnp.full((_LANES,), t0 + j, jnp.int32)
                for c in range(d // _LANES):
                    f_v = tok_iota + c * _LANES
                    val = plsc.load_gather(bufs[slot], [j_v, f_v, lane_v])
                    p = plsc.load_gather(pos_v, [f_v, t_v])
                    plsc.store_scatter(outc_v, [f_v, t_v], val + p)

        def group(g, prev_vec):
            vec = idx_v[pl.ds(g * _LANES, _LANES)]
            for q in range(group_chunks):
                slot = q % _SLOTS
                if q >= _SLOTS:
                    pq = q - _SLOTS
                    wait_chunk(slot)
                    process_chunk(slot, vec, pq * _CHUNK,
                                  g * _LANES + pq * _CHUNK)
                else:
                    pq = q + group_chunks - _SLOTS

                    @pl.when(g >= 1)
                    def _():
                        wait_chunk(slot)
                        process_chunk(slot, prev_vec, pq * _CHUNK,
                                      (g - 1) * _LANES + pq * _CHUNK)
                fire_chunk(slot, vec, q * _CHUNK)
            return vec

        last_vec = lax.fori_loop(0, b_per_w // _LANES, group,
                                 jnp.zeros((_LANES,), jnp.int32))
        for q in range(group_chunks - _SLOTS, group_chunks):
            slot = q % _SLOTS
            wait_chunk(slot)
            process_chunk(slot, last_vec, q * _CHUNK,
                          (b_per_w // _LANES - 1) * _LANES + q * _CHUNK)

        pltpu.sync_copy(outc_v, out_hbm.at[b_i, :, pl.ds(s_off, b_per_w)])

    return gather_add


def kernel(token_ids, seq_len, token_table, pos_table):
    b, s = token_ids.shape
    _, d = token_table.shape
    table_t = token_table.T
    pos_t = pos_table.T
    if pos_table.shape[0] == s:
        pos_sl = pos_t
    else:
        pos_sl = lax.dynamic_slice(pos_t, (0, seq_len - s), (d, s))
    flat_ids = token_ids.reshape(b * s).astype(jnp.int32)
    out_t = _build_gather_add(b * s, b, s, d)(flat_ids, table_t, pos_sl)
    return jnp.transpose(out_t, (0, 2, 1))


# speedup vs baseline: 1.0083x; 1.0083x over previous
import functools

import jax
import jax.numpy as jnp
from jax import lax
from jax.experimental import pallas as pl
from jax.experimental.pallas import tpu as pltpu
from jax.experimental.pallas import tpu_sc as plsc

_NUM_CORES = 2
_NUM_SUBCORES = 16
_LANES = 16
_NW = _NUM_CORES * _NUM_SUBCORES
_TILE_LANES = 128
_CHUNK = 1
_SLOTS = 8


@functools.lru_cache(maxsize=None)
def _build_gather_add(n_tokens: int, batch: int, seq_len_s: int, d: int):
    b_per_w = n_tokens // _NW
    chunks_per_row = seq_len_s // b_per_w
    n_chunks = b_per_w // _CHUNK
    group_chunks = _LANES // _CHUNK

    mesh = plsc.VectorSubcoreMesh(
        core_axis_name="c", subcore_axis_name="s",
        num_cores=_NUM_CORES, num_subcores=_NUM_SUBCORES)

    @functools.partial(
        pl.kernel,
        mesh=mesh,
        compiler_params=pltpu.CompilerParams(needs_layout_passes=False),
        out_type=jax.ShapeDtypeStruct((batch, d, seq_len_s), jnp.float32),
        scratch_types=[
            pltpu.VMEM((b_per_w,), jnp.int32),
            pltpu.VMEM((d, b_per_w), jnp.float32),
            pltpu.VMEM((d, b_per_w), jnp.float32),
            [pltpu.VMEM((_CHUNK, d, _TILE_LANES), jnp.float32)] * _SLOTS,
            [pltpu.SemaphoreType.DMA] * _SLOTS,
        ],
    )
    def gather_add(ids_hbm, table_hbm, pos_hbm, out_hbm, idx_v, pos_v,
                   outc_v, bufs, sems):
        wid = lax.axis_index("s") * _NUM_CORES + lax.axis_index("c")
        base = wid * b_per_w
        b_i = wid // chunks_per_row
        s_off = pl.multiple_of(
            lax.rem(wid, chunks_per_row) * b_per_w, _TILE_LANES)
        pltpu.sync_copy(ids_hbm.at[pl.ds(base, b_per_w)], idx_v)
        pltpu.sync_copy(pos_hbm.at[:, pl.ds(s_off, b_per_w)], pos_v)

        tok_iota = lax.iota(jnp.int32, _LANES)

        def fire_chunk(slot, vec, lane0):
            for j in range(_CHUNK):
                col0 = pl.multiple_of(
                    lax.bitwise_and(vec[lane0 + j], -_TILE_LANES),
                    _TILE_LANES)
                pltpu.async_copy(
                    table_hbm.at[:, pl.ds(col0, _TILE_LANES)],
                    bufs[slot].at[j], sems[slot])

        def wait_chunk(slot):
            for j in range(_CHUNK):
                pltpu.make_async_copy(
                    table_hbm.at[:, pl.ds(0, _TILE_LANES)],
                    bufs[slot].at[j], sems[slot]).wait()

        def process_chunk(slot, vec, lane0, t0):
            for j in range(_CHUNK):
                lane_v = jnp.full(
                    (_LANES,), lax.bitwise_and(vec[lane0 + j], _TILE_LANES - 1),
                    jnp.int32)
                j_v = jnp.full((_LANES,), j, jnp.int32)
                t_v = jnp.full((_LANES,), t0 + j, jnp.int32)
                for c in range(d // _LANES):
                    f_v = tok_iota + c * _LANES
                    val = plsc.load_gather(bufs[slot], [j_v, f_v, lane_v])
                    p = plsc.load_gather(pos_v, [f_v, t_v])
                    plsc.store_scatter(outc_v, [f_v, t_v], val + p)

        def group(g, prev_vec):
            vec = idx_v[pl.ds(g * _LANES, _LANES)]
            for q in range(group_chunks):
                slot = q % _SLOTS
                if q >= _SLOTS:
                    pq = q - _SLOTS
                    wait_chunk(slot)
                    process_chunk(slot, vec, pq * _CHUNK,
                                  g * _LANES + pq * _CHUNK)
                else:
                    pq = q + group_chunks - _SLOTS

                    @pl.when(g >= 1)
                    def _():
                        wait_chunk(slot)
                        process_chunk(slot, prev_vec, pq * _CHUNK,
                                      (g - 1) * _LANES + pq * _CHUNK)
                fire_chunk(slot, vec, q * _CHUNK)
            return vec

        last_vec = lax.fori_loop(0, b_per_w // _LANES, group,
                                 jnp.zeros((_LANES,), jnp.int32))
        for q in range(group_chunks - _SLOTS, group_chunks):
            slot = q % _SLOTS
            wait_chunk(slot)
            process_chunk(slot, last_vec, q * _CHUNK,
                          (b_per_w // _LANES - 1) * _LANES + q * _CHUNK)

        pltpu.sync_copy(outc_v, out_hbm.at[b_i, :, pl.ds(s_off, b_per_w)])

    return gather_add


def kernel(token_ids, seq_len, token_table, pos_table):
    b, s = token_ids.shape
    _, d = token_table.shape
    table_t = token_table.T
    pos_t = pos_table.T
    if pos_table.shape[0] == s:
        pos_sl = pos_t
    else:
        pos_sl = lax.dynamic_slice(pos_t, (0, seq_len - s), (d, s))
    flat_ids = token_ids.reshape(b * s).astype(jnp.int32)
    out_t = _build_gather_add(b * s, b, s, d)(flat_ids, table_t, pos_sl)
    return jnp.transpose(out_t, (0, 2, 1))
